# Initial kernel scaffold; baseline (speedup 1.0000x reference)
import jax
import jax.numpy as jnp
from jax.experimental import pallas as pl


def kernel(x, edge_index, edge_attr, params):
    raise NotImplementedError("write your pallas kernel here")


# speedup vs baseline: 1.7603x; 1.7603x over previous
import jax
import jax.numpy as jnp
from jax import lax
from jax.experimental import pallas as pl
from jax.experimental.pallas import tpu as pltpu
from jax.experimental.pallas import tpu_sc as plsc
from jax.experimental.layout import Layout, with_layout_constraint

N_NODES = 10000
N_EDGES = 320000
D = 32

NC = 2
NS = 16
NW = NC * NS
EPW = N_EDGES // NW
GC = 400
SCC = 400

_f32 = jnp.float32


def _ln(t, g, b):
    mu = jnp.mean(t, axis=-1, keepdims=True)
    var = jnp.mean((t - mu) * (t - mu), axis=-1, keepdims=True)
    return (t - mu) * lax.rsqrt(var + 1e-5) * g + b


def _mm(a, w):
    return jnp.dot(a, w, preferred_element_type=_f32)


def _node_enc_body(x, w0, b0, w1, b1, g, b, o):
    t = jnp.maximum(_mm(x[...], w0[...]) + b0[...], 0.0)
    t = _mm(t, w1[...]) + b1[...]
    o[...] = _ln(t, g[...], b[...])


def _edge_enc_body(a, w0, b0, w1, b1, g, b, o):
    t = jnp.maximum(_mm(a[...], w0[...]) + b0[...], 0.0)
    t = _mm(t, w1[...]) + b1[...]
    o[...] = _ln(t, g[...], b[...])


def _edge_mlp_body(gd, gs, e, wi, wj, we, b0, w1, b1, w2, b2, w3, b3, w4, b4,
                   g, b, m_o, e_o):
    ev = e[...]
    t = jnp.maximum(_mm(gd[...], wi[...]) + _mm(gs[...], wj[...])
                    + _mm(ev, we[...]) + b0[...], 0.0)
    t = jnp.maximum(_mm(t, w1[...]) + b1[...], 0.0)
    t = jnp.maximum(_mm(t, w2[...]) + b2[...], 0.0)
    t = jnp.maximum(_mm(t, w3[...]) + b3[...], 0.0)
    t = _mm(t, w4[...]) + b4[...]
    m = _ln(t, g[...], b[...])
    m_o[...] = m
    e_o[...] = ev + m


def _node_mlp_body(p, h, w0, b0, w1, b1, w2, b2, w3, b3, w4, b4, g, b, h_o):
    agg = p[:N_NODES] + p[ACC_ROWS:ACC_ROWS + N_NODES]
    t = jnp.maximum(_mm(agg, w0[...]) + b0[...], 0.0)
    t = jnp.maximum(_mm(t, w1[...]) + b1[...], 0.0)
    t = jnp.maximum(_mm(t, w2[...]) + b2[...], 0.0)
    t = jnp.maximum(_mm(t, w3[...]) + b3[...], 0.0)
    t = _mm(t, w4[...]) + b4[...]
    h_o[...] = h[...] + _ln(t, g[...], b[...])


def _node_mlp_dec_body(p, h, w0, b0, w1, b1, w2, b2, w3, b3, w4, b4, g, b,
                       dw0, db0, dw1, db1, o):
    agg = p[:N_NODES] + p[ACC_ROWS:ACC_ROWS + N_NODES]
    t = jnp.maximum(_mm(agg, w0[...]) + b0[...], 0.0)
    t = jnp.maximum(_mm(t, w1[...]) + b1[...], 0.0)
    t = jnp.maximum(_mm(t, w2[...]) + b2[...], 0.0)
    t = jnp.maximum(_mm(t, w3[...]) + b3[...], 0.0)
    t = _mm(t, w4[...]) + b4[...]
    hn = h[...] + _ln(t, g[...], b[...])
    t = jnp.maximum(_mm(hn, dw0[...]) + db0[...], 0.0)
    o[...] = _mm(t, dw1[...]) + db1[...]


def _full(shape):
    return pl.BlockSpec(shape, lambda *_: tuple(0 for _ in shape))


def _rows(shape):
    n = len(shape)
    return pl.BlockSpec(shape, lambda i: (i,) + tuple(0 for _ in range(n - 1)))


def _tc_call(body, grid, in_specs, out_specs, out_shape):
    return pl.pallas_call(
        body, grid=grid, in_specs=in_specs, out_specs=out_specs,
        out_shape=out_shape)


_MESH = plsc.VectorSubcoreMesh(
    core_axis_name="c", subcore_axis_name="s", num_cores=NC, num_subcores=NS)


def _sc_gather_body(h_hbm, dst_hbm, src_hbm, gd_hbm, gs_hbm,
                    idx_d, idx_s, rows_d, rows_s, sem_d, sem_s):
    cid = lax.axis_index("c")
    sid = lax.axis_index("s")
    wid = sid * NC + cid
    base0 = wid * EPW

    def step(k, carry):
        base = base0 + k * GC
        pltpu.sync_copy(dst_hbm.at[pl.ds(base, GC)], idx_d)
        pltpu.sync_copy(src_hbm.at[pl.ds(base, GC)], idx_s)
        cp_d = pltpu.async_copy(h_hbm.at[idx_d], rows_d, sem_d)
        cp_s = pltpu.async_copy(h_hbm.at[idx_s], rows_s, sem_s)
        cp_d.wait()
        cp_s.wait()
        pltpu.sync_copy(rows_d, gd_hbm.at[pl.ds(base, GC)])
        pltpu.sync_copy(rows_s, gs_hbm.at[pl.ds(base, GC)])
        return carry

    lax.fori_loop(0, EPW // GC, step, 0)


_sc_gather = pl.kernel(
    _sc_gather_body,
    out_type=(jax.ShapeDtypeStruct((N_EDGES, D), _f32),
              jax.ShapeDtypeStruct((N_EDGES, D), _f32)),
    mesh=_MESH,
    scratch_types=[
        pltpu.VMEM((GC,), jnp.int32),
        pltpu.VMEM((GC,), jnp.int32),
        pltpu.VMEM((GC, D), _f32),
        pltpu.VMEM((GC, D), _f32),
        pltpu.SemaphoreType.DMA,
        pltpu.SemaphoreType.DMA,
    ],
)


SH_E = N_EDGES // NC
RNG = 640
ACC_ROWS = NS * RNG
SCC = 800
GB = 80


def _r2(v):
    return v.reshape(1, -1)


def kernel(x, edge_index, edge_attr, params):
    src = edge_index[0]
    dst = edge_index[1]

    (enw0, enb0), (enw1, enb1) = params['enc_node']
    eng, enb = params['enc_node_ln']
    (eew0, eeb0), (eew1, eeb1) = params['enc_edge']
    eeg, eeb = params['enc_edge_ln']
    (dw0, db0), (dw1, db1) = params['dec']

    h = _tc_call(
        _node_enc_body, (1,),
        [_full((N_NODES, 128)), _full((128, D)), _full((1, D)),
         _full((D, D)), _full((1, D)), _full((1, D)), _full((1, D))],
        _full((N_NODES, D)),
        jax.ShapeDtypeStruct((N_NODES, D), _f32),
    )(x, enw0, _r2(enb0), enw1, _r2(enb1), _r2(eng), _r2(enb))

    BE = 8000
    e = _tc_call(
        _edge_enc_body, (N_EDGES // BE,),
        [_rows((BE, 4)), _full((4, D)), _full((1, D)),
         _full((D, D)), _full((1, D)), _full((1, D)), _full((1, D))],
        _rows((BE, D)),
        jax.ShapeDtypeStruct((N_EDGES, D), _f32),
    )(edge_attr, eew0, _r2(eeb0), eew1, _r2(eeb1), _r2(eeg), _r2(eeb))

    edge_mlp_call = _tc_call(
        _edge_mlp_body, (N_EDGES // BE,),
        [_rows((BE, D))] * 3
        + [_full((D, D)), _full((D, D)), _full((D, D)), _full((1, D)),
           _full((D, D)), _full((1, D)), _full((D, D)), _full((1, D)),
           _full((D, D)), _full((1, D)), _full((D, D)), _full((1, D)),
           _full((1, D)), _full((1, D))],
        (_rows((BE, D)), _rows((BE, D))),
        (jax.ShapeDtypeStruct((N_EDGES, D), _f32),
         jax.ShapeDtypeStruct((N_EDGES, D), _f32)),
    )

    node_specs = [_full((NC * ACC_ROWS, D)), _full((N_NODES, D)),
                  _full((D, D)), _full((1, D)), _full((D, D)), _full((1, D)),
                  _full((D, D)), _full((1, D)), _full((D, D)), _full((1, D)),
                  _full((D, D)), _full((1, D)), _full((1, D)), _full((1, D))]

    node_mlp_call = _tc_call(
        _node_mlp_body, (1,), node_specs,
        _full((N_NODES, D)),
        jax.ShapeDtypeStruct((N_NODES, D), _f32),
    )

    node_dec_call = _tc_call(
        _node_mlp_dec_body, (1,),
        node_specs + [_full((D, D)), _full((1, D)), _full((D, 3)),
                      _full((1, 3))],
        _full((N_NODES, 3)),
        jax.ShapeDtypeStruct((N_NODES, 3), _f32),
    )

    out = None
    for li, lp in enumerate(params['layers']):
        wcat, b0 = lp['edge_mlp'][0]
        wi, wj, we = wcat[:D], wcat[D:2 * D], wcat[2 * D:]
        (w1, b1), (w2, b2), (w3, b3), (w4, b4) = lp['edge_mlp'][1:]
        eg, eb = lp['edge_ln']
        (nw0, nb0), (nw1, nb1), (nw2, nb2), (nw3, nb3), (nw4, nb4) = lp['node_mlp']
        ng, nb = lp['node_ln']

        h_lin = with_layout_constraint(
            h, Layout(major_to_minor=(0, 1), tiling=((16,),)))
        gd, gs = _sc_gather(h_lin, dst, src)
        m, e = edge_mlp_call(
            gd, gs, e, wi, wj, we, _r2(b0), w1, _r2(b1), w2, _r2(b2),
            w3, _r2(b3), w4, _r2(b4), _r2(eg), _r2(eb))
        seg = jax.ops.segment_sum(m, dst, num_segments=N_NODES)
        seg = jnp.pad(seg, ((0, ACC_ROWS - N_NODES), (0, 0)))
        partials = jnp.concatenate(
            [seg, jnp.zeros((ACC_ROWS, D), _f32)], axis=0)
        node_args = (partials, h, nw0, _r2(nb0), nw1, _r2(nb1), nw2, _r2(nb2),
                     nw3, _r2(nb3), nw4, _r2(nb4), _r2(ng), _r2(nb))
        if li == len(params['layers']) - 1:
            out = node_dec_call(*node_args, dw0, _r2(db0), dw1, _r2(db1))
        else:
            h = node_mlp_call(*node_args)

    return out
